# baseline (speedup 1.0000x reference)
import functools

import jax
import jax.numpy as jnp
from jax import lax
from jax.experimental import pallas as pl
from jax.experimental.pallas import tpu as pltpu
from jax.experimental.pallas import tpu_sc as plsc

N_NODES = 10000
N_EDGES = 320000
D = 128

NC = 2
NS = 16
NW = NC * NS
GC = 80
ROW_CHUNKS = N_NODES // GC

EB = 2000
NB = 2000

_SC_PARAMS = pltpu.CompilerParams(use_tc_tiling_on_sc=True,
                                  needs_layout_passes=True)


def _sc_mesh():
    return plsc.VectorSubcoreMesh(core_axis_name="c", subcore_axis_name="s",
                                  num_cores=NC, num_subcores=NS)


def _ln(y, g, b):
    mu = jnp.mean(y, axis=-1, keepdims=True)
    yc = y - mu
    var = jnp.mean(yc * yc, axis=-1, keepdims=True)
    return yc / jnp.sqrt(var + 1e-5) * g + b


def _dot(a, b):
    return jnp.dot(a, b, preferred_element_type=jnp.float32)


def _full(shape):
    return pl.BlockSpec(shape, lambda i: tuple(0 for _ in shape))


def _rows(shape):
    return pl.BlockSpec(shape, lambda i: (i, 0))


def _mlp_ln_kernel(x_ref, w1, b1, w2, b2, w3, b3, g, beta, o_ref):
    x1 = jax.nn.relu(_dot(x_ref[...], w1[...]) + b1[...])
    x2 = jax.nn.relu(_dot(x1, w2[...]) + b2[...])
    y = _dot(x2, w3[...]) + b3[...]
    o_ref[...] = _ln(y, g[...], beta[...])


def _mlp_ln(x, p, block_rows):
    n, din = x.shape
    w1, w2, w3 = p["fc1"]["w"], p["fc2"]["w"], p["fc3"]["w"]
    dout = w3.shape[1]
    args = (x, w1, p["fc1"]["b"][None, :], w2, p["fc2"]["b"][None, :],
            w3, p["fc3"]["b"][None, :], p["ln"]["g"][None, :], p["ln"]["b"][None, :])
    return pl.pallas_call(
        _mlp_ln_kernel,
        grid=(n // block_rows,),
        in_specs=[_rows((block_rows, din)), _full(w1.shape), _full((1, D)),
                  _full(w2.shape), _full((1, D)), _full(w3.shape), _full((1, dout)),
                  _full((1, dout)), _full((1, dout))],
        out_specs=_rows((block_rows, dout)),
        out_shape=jax.ShapeDtypeStruct((n, dout), jnp.float32),
    )(*args)


def _dec_kernel(x_ref, w1, b1, w2, b2, w3, b3, o_ref):
    x1 = jax.nn.relu(_dot(x_ref[...], w1[...]) + b1[...])
    x2 = jax.nn.relu(_dot(x1, w2[...]) + b2[...])
    o_ref[...] = _dot(x2, w3[...]) + b3[...]


def _decoder(x, p, block_rows):
    n, din = x.shape
    dout = p["fc3"]["w"].shape[1]
    return pl.pallas_call(
        _dec_kernel,
        grid=(n // block_rows,),
        in_specs=[_rows((block_rows, din)), _full((din, D)), _full((1, D)),
                  _full((D, D)), _full((1, D)), _full((D, dout)), _full((1, dout))],
        out_specs=_rows((block_rows, dout)),
        out_shape=jax.ShapeDtypeStruct((n, dout), jnp.float32),
    )(x, p["fc1"]["w"], p["fc1"]["b"][None, :], p["fc2"]["w"], p["fc2"]["b"][None, :],
      p["fc3"]["w"], p["fc3"]["b"][None, :])


def _edge_mlp_kernel(he_ref, g2_ref, w1, b1, w2, b2, w3, b3, g, beta,
                     ho_ref, u_ref):
    he = he_ref[...]
    x = jnp.concatenate([he, g2_ref[0], g2_ref[1]], axis=-1)
    x1 = jax.nn.relu(_dot(x, w1[...]) + b1[...])
    x2 = jax.nn.relu(_dot(x1, w2[...]) + b2[...])
    y = _dot(x2, w3[...]) + b3[...]
    u = _ln(y, g[...], beta[...])
    u_ref[...] = u
    ho_ref[...] = he + u


def _edge_mlp(h_edges, gsr, w1, b1, w2, b2, w3, b3, g, beta):
    return pl.pallas_call(
        _edge_mlp_kernel,
        grid=(N_EDGES // EB,),
        in_specs=[_rows((EB, D)),
                  pl.BlockSpec((2, EB, D), lambda i: (0, i, 0)),
                  _full((3 * D, D)), _full((1, D)), _full((D, D)), _full((1, D)),
                  _full((D, D)), _full((1, D)), _full((1, D)), _full((1, D))],
        out_specs=(_rows((EB, D)), _rows((EB, D))),
        out_shape=(jax.ShapeDtypeStruct((N_EDGES, D), jnp.float32),
                   jax.ShapeDtypeStruct((N_EDGES, D), jnp.float32)),
    )(h_edges, gsr, w1, b1, w2, b2, w3, b3, g, beta)


def _node_mlp_kernel(h_ref, p_ref, w1, b1, w2, b2, w3, b3, g, beta, ho_ref):
    h = h_ref[...]
    x = jnp.concatenate([h, p_ref[0] + p_ref[1]], axis=-1)
    x1 = jax.nn.relu(_dot(x, w1[...]) + b1[...])
    x2 = jax.nn.relu(_dot(x1, w2[...]) + b2[...])
    y = _dot(x2, w3[...]) + b3[...]
    ho_ref[...] = h + _ln(y, g[...], beta[...])


def _node_mlp(h, p, w1, b1, w2, b2, w3, b3, g, beta):
    return pl.pallas_call(
        _node_mlp_kernel,
        grid=(N_NODES // NB,),
        in_specs=[_rows((NB, D)),
                  pl.BlockSpec((2, NB, D), lambda i: (0, i, 0)),
                  _full((2 * D, D)), _full((1, D)), _full((D, D)), _full((1, D)),
                  _full((D, D)), _full((1, D)), _full((1, D)), _full((1, D))],
        out_specs=_rows((NB, D)),
        out_shape=jax.ShapeDtypeStruct((N_NODES, D), jnp.float32),
    )(h, p, w1, b1, w2, b2, w3, b3, g, beta)


EPT = N_EDGES // NS
GNCH = EPT // GC


def _sc_gather(h_nodes, idx_flat):
    @functools.partial(
        pl.kernel,
        out_type=jax.ShapeDtypeStruct((2, N_EDGES, D), jnp.float32),
        mesh=_sc_mesh(),
        compiler_params=_SC_PARAMS,
        scratch_types=[
            pltpu.VMEM((GC,), jnp.int32),
            pltpu.VMEM((GC,), jnp.int32),
            pltpu.VMEM((GC, D), jnp.float32),
            pltpu.VMEM((GC, D), jnp.float32),
            pltpu.VMEM((GC, D), jnp.float32),
            pltpu.VMEM_SHARED((N_NODES, D), jnp.float32),
            pltpu.SemaphoreType.DMA,
            pltpu.SemaphoreType.DMA,
            pltpu.SemaphoreType.DMA,
        ],
    )
    def k(tbl_hbm, idx_hbm, g_hbm,
          i0_v, i1_v, r0_v, r1_v, stage_v, tbl_sh, gsem, osem0, osem1):
        cid = lax.axis_index("c")
        sid = lax.axis_index("s")

        def tload(j, carry):
            c = sid + j * NS

            @pl.when(c < ROW_CHUNKS)
            def _():
                pltpu.sync_copy(tbl_hbm.at[pl.ds(c * GC, GC)], stage_v)
                pltpu.sync_copy(stage_v, tbl_sh.at[pl.ds(c * GC, GC)])
            return carry

        lax.fori_loop(0, (ROW_CHUNKS + NS - 1) // NS, tload, 0)
        plsc.subcore_barrier()

        bufs = ((i0_v, r0_v, osem0), (i1_v, r1_v, osem1))

        def pair(j, carry):
            for bi, (iv, rv, osem) in enumerate(bufs):
                base = sid * EPT + (2 * j + bi) * GC

                @pl.when(j > 0)
                def _():
                    pltpu.make_async_copy(
                        rv, g_hbm.at[cid, pl.ds(0, GC)], osem).wait()

                pltpu.sync_copy(
                    idx_hbm.at[pl.ds(cid * N_EDGES + base, GC)], iv)
                pltpu.async_copy(tbl_sh.at[iv], rv, gsem).wait()
                pltpu.async_copy(rv, g_hbm.at[cid, pl.ds(base, GC)], osem)
            return carry

        lax.fori_loop(0, GNCH // 2, pair, 0)
        for iv, rv, osem in bufs:
            pltpu.make_async_copy(rv, g_hbm.at[cid, pl.ds(0, GC)], osem).wait()

    return k(h_nodes, idx_flat)


def _sc_scatter(u, ridx, zeros_blk):
    epw = N_EDGES // NW
    nchunk = epw // GC

    @functools.partial(
        pl.kernel,
        out_type=jax.ShapeDtypeStruct((NC, N_NODES, D), jnp.float32),
        mesh=_sc_mesh(),
        compiler_params=_SC_PARAMS,
        scratch_types=[
            pltpu.VMEM((GC, D), jnp.float32),
            pltpu.VMEM((GC,), jnp.int32),
            pltpu.VMEM((GC, D), jnp.float32),
            pltpu.VMEM((GC,), jnp.int32),
            pltpu.VMEM_SHARED((N_NODES, D), jnp.float32),
            pltpu.SemaphoreType.DMA,
            pltpu.SemaphoreType.DMA,
        ],
    )
    def k(u_hbm, ri_hbm, z_hbm, p_hbm, u_v, ri_v, u2_v, ri2_v, acc_sh,
          lsem0, lsem1):
        cid = lax.axis_index("c")
        sid = lax.axis_index("s")
        wid = sid * NC + cid

        pltpu.sync_copy(z_hbm, u_v)

        def zloop(j, carry):
            c = sid + j * NS

            @pl.when(c < ROW_CHUNKS)
            def _():
                pltpu.sync_copy(u_v, acc_sh.at[pl.ds(c * GC, GC)])
            return carry

        lax.fori_loop(0, (ROW_CHUNKS + NS - 1) // NS, zloop, 0)
        plsc.subcore_barrier()

        ebase = wid * epw
        pltpu.sync_copy(ri_hbm.at[pl.ds(ebase, GC)], ri_v)
        pltpu.sync_copy(u_hbm.at[pl.ds(ebase, GC)], u_v)

        def pair(j, carry):
            b1 = ebase + (2 * j + 1) * GC
            c1u = pltpu.async_copy(u_hbm.at[pl.ds(b1, GC)], u2_v, lsem0)
            c1r = pltpu.async_copy(ri_hbm.at[pl.ds(b1, GC)], ri2_v, lsem1)
            pltpu.sync_copy(u_v, acc_sh.at[ri_v], add=True)
            c1u.wait()
            c1r.wait()
            b2 = ebase + (2 * j + 2) * GC
            c2u = pltpu.async_copy(u_hbm.at[pl.ds(b2, GC)], u_v, lsem0)
            c2r = pltpu.async_copy(ri_hbm.at[pl.ds(b2, GC)], ri_v, lsem1)
            pltpu.sync_copy(u2_v, acc_sh.at[ri2_v], add=True)
            c2u.wait()
            c2r.wait()
            return carry

        lax.fori_loop(0, (nchunk - 1) // 2, pair, 0)
        pltpu.sync_copy(u_v, acc_sh.at[ri_v], add=True)
        plsc.subcore_barrier()

        def oloop(j, carry):
            c = sid + j * NS

            @pl.when(c < ROW_CHUNKS)
            def _():
                pltpu.sync_copy(acc_sh.at[pl.ds(c * GC, GC)], u_v)
                pltpu.sync_copy(u_v, p_hbm.at[cid, pl.ds(c * GC, GC)])
            return carry

        lax.fori_loop(0, (ROW_CHUNKS + NS - 1) // NS, oloop, 0)

    return k(u, ridx, zeros_blk)


def kernel(node_features, edge_features, edge_index, params):
    idx2 = edge_index.astype(jnp.int32)
    idx_flat = idx2.reshape(-1)
    receiver = idx2[1]
    zeros_blk = jnp.zeros((GC, D), jnp.float32)

    h_nodes = _mlp_ln(node_features, params["node_enc"], NB)
    h_edges = _mlp_ln(edge_features, params["edge_enc"], EB)

    layers = params["layers"]

    def stack(fn):
        return jnp.stack([fn(lp) for lp in layers])

    lw = {
        "ew1": stack(lambda lp: lp["edge_mlp"]["fc1"]["w"]),
        "eb1": stack(lambda lp: lp["edge_mlp"]["fc1"]["b"]),
        "ew2": stack(lambda lp: lp["edge_mlp"]["fc2"]["w"]),
        "eb2": stack(lambda lp: lp["edge_mlp"]["fc2"]["b"]),
        "ew3": stack(lambda lp: lp["edge_mlp"]["fc3"]["w"]),
        "eb3": stack(lambda lp: lp["edge_mlp"]["fc3"]["b"]),
        "eg": stack(lambda lp: lp["edge_mlp"]["ln"]["g"]),
        "ebt": stack(lambda lp: lp["edge_mlp"]["ln"]["b"]),
        "nw1": stack(lambda lp: lp["node_mlp"]["fc1"]["w"]),
        "nb1": stack(lambda lp: lp["node_mlp"]["fc1"]["b"]),
        "nw2": stack(lambda lp: lp["node_mlp"]["fc2"]["w"]),
        "nb2": stack(lambda lp: lp["node_mlp"]["fc2"]["b"]),
        "nw3": stack(lambda lp: lp["node_mlp"]["fc3"]["w"]),
        "nb3": stack(lambda lp: lp["node_mlp"]["fc3"]["b"]),
        "ng": stack(lambda lp: lp["node_mlp"]["ln"]["g"]),
        "nbt": stack(lambda lp: lp["node_mlp"]["ln"]["b"]),
    }

    def step(carry, w):
        h_nodes, h_edges = carry
        gsr = _sc_gather(h_nodes, idx_flat)
        h_edges, u = _edge_mlp(h_edges, gsr, w["ew1"], w["eb1"][None, :],
                               w["ew2"], w["eb2"][None, :], w["ew3"],
                               w["eb3"][None, :], w["eg"][None, :],
                               w["ebt"][None, :])
        p = _sc_scatter(u, receiver, zeros_blk)
        h_nodes = _node_mlp(h_nodes, p, w["nw1"], w["nb1"][None, :],
                            w["nw2"], w["nb2"][None, :], w["nw3"],
                            w["nb3"][None, :], w["ng"][None, :],
                            w["nbt"][None, :])
        return (h_nodes, h_edges), None

    (h_nodes, h_edges), _ = lax.scan(step, (h_nodes, h_edges), lw)
    return _decoder(h_nodes, params["decoder"], NB)


# speedup vs baseline: 1.0908x; 1.0908x over previous
import functools

import jax
import jax.numpy as jnp
from jax import lax
from jax.experimental import pallas as pl
from jax.experimental.pallas import tpu as pltpu
from jax.experimental.pallas import tpu_sc as plsc

N_NODES = 10000
N_EDGES = 320000
D = 128

NC = 2
NS = 16
NW = NC * NS
GC = 80
ROW_CHUNKS = N_NODES // GC

EB = 2000
NB = 2000

_SC_PARAMS = None


def _sc_mesh():
    return plsc.VectorSubcoreMesh(core_axis_name="c", subcore_axis_name="s",
                                  num_cores=NC, num_subcores=NS)


def _ln(y, g, b):
    mu = jnp.mean(y, axis=-1, keepdims=True)
    yc = y - mu
    var = jnp.mean(yc * yc, axis=-1, keepdims=True)
    return yc / jnp.sqrt(var + 1e-5) * g + b


def _dot(a, b):
    return jnp.dot(a, b, preferred_element_type=jnp.float32)


def _full(shape):
    return pl.BlockSpec(shape, lambda i: tuple(0 for _ in shape))


def _rows(shape):
    return pl.BlockSpec(shape, lambda i: (i, 0))


def _mlp_ln_kernel(x_ref, w1, b1, w2, b2, w3, b3, g, beta, o_ref):
    x1 = jax.nn.relu(_dot(x_ref[...], w1[...]) + b1[...])
    x2 = jax.nn.relu(_dot(x1, w2[...]) + b2[...])
    y = _dot(x2, w3[...]) + b3[...]
    o_ref[...] = _ln(y, g[...], beta[...])


def _mlp_ln(x, p, block_rows):
    n, din = x.shape
    w1, w2, w3 = p["fc1"]["w"], p["fc2"]["w"], p["fc3"]["w"]
    dout = w3.shape[1]
    args = (x, w1, p["fc1"]["b"][None, :], w2, p["fc2"]["b"][None, :],
            w3, p["fc3"]["b"][None, :], p["ln"]["g"][None, :], p["ln"]["b"][None, :])
    return pl.pallas_call(
        _mlp_ln_kernel,
        grid=(n // block_rows,),
        in_specs=[_rows((block_rows, din)), _full(w1.shape), _full((1, D)),
                  _full(w2.shape), _full((1, D)), _full(w3.shape), _full((1, dout)),
                  _full((1, dout)), _full((1, dout))],
        out_specs=_rows((block_rows, dout)),
        out_shape=jax.ShapeDtypeStruct((n, dout), jnp.float32),
    )(*args)


def _dec_kernel(x_ref, w1, b1, w2, b2, w3, b3, o_ref):
    x1 = jax.nn.relu(_dot(x_ref[...], w1[...]) + b1[...])
    x2 = jax.nn.relu(_dot(x1, w2[...]) + b2[...])
    o_ref[...] = _dot(x2, w3[...]) + b3[...]


def _decoder(x, p, block_rows):
    n, din = x.shape
    dout = p["fc3"]["w"].shape[1]
    return pl.pallas_call(
        _dec_kernel,
        grid=(n // block_rows,),
        in_specs=[_rows((block_rows, din)), _full((din, D)), _full((1, D)),
                  _full((D, D)), _full((1, D)), _full((D, dout)), _full((1, dout))],
        out_specs=_rows((block_rows, dout)),
        out_shape=jax.ShapeDtypeStruct((n, dout), jnp.float32),
    )(x, p["fc1"]["w"], p["fc1"]["b"][None, :], p["fc2"]["w"], p["fc2"]["b"][None, :],
      p["fc3"]["w"], p["fc3"]["b"][None, :])


def _edge_mlp_kernel(he_ref, g2_ref, w1, b1, w2, b2, w3, b3, g, beta,
                     ho_ref, u_ref):
    he = he_ref[...]
    x = jnp.concatenate([he, g2_ref[0], g2_ref[1]], axis=-1)
    x1 = jax.nn.relu(_dot(x, w1[...]) + b1[...])
    x2 = jax.nn.relu(_dot(x1, w2[...]) + b2[...])
    y = _dot(x2, w3[...]) + b3[...]
    u = _ln(y, g[...], beta[...])
    u_ref[...] = u
    ho_ref[...] = he + u


def _edge_mlp(h_edges, gsr, w1, b1, w2, b2, w3, b3, g, beta):
    n_e = h_edges.shape[0]
    return pl.pallas_call(
        _edge_mlp_kernel,
        grid=(n_e // EB,),
        in_specs=[_rows((EB, D)),
                  pl.BlockSpec((2, EB, D), lambda i: (0, i, 0)),
                  _full((3 * D, D)), _full((1, D)), _full((D, D)), _full((1, D)),
                  _full((D, D)), _full((1, D)), _full((1, D)), _full((1, D))],
        out_specs=(_rows((EB, D)), _rows((EB, D))),
        out_shape=(jax.ShapeDtypeStruct((n_e, D), jnp.float32),
                   jax.ShapeDtypeStruct((n_e, D), jnp.float32)),
    )(h_edges, gsr, w1, b1, w2, b2, w3, b3, g, beta)


def _node_mlp_kernel(h_ref, pa_ref, pb_ref, w1, b1, w2, b2, w3, b3, g, beta,
                     ho_ref):
    h = h_ref[...]
    agg = pa_ref[0] + pa_ref[1] + pb_ref[0] + pb_ref[1]
    x = jnp.concatenate([h, agg], axis=-1)
    x1 = jax.nn.relu(_dot(x, w1[...]) + b1[...])
    x2 = jax.nn.relu(_dot(x1, w2[...]) + b2[...])
    y = _dot(x2, w3[...]) + b3[...]
    ho_ref[...] = h + _ln(y, g[...], beta[...])


def _node_mlp(h, pa, pb, w1, b1, w2, b2, w3, b3, g, beta):
    p2 = pl.BlockSpec((2, NB, D), lambda i: (0, i, 0))
    return pl.pallas_call(
        _node_mlp_kernel,
        grid=(N_NODES // NB,),
        in_specs=[_rows((NB, D)), p2, p2,
                  _full((2 * D, D)), _full((1, D)), _full((D, D)), _full((1, D)),
                  _full((D, D)), _full((1, D)), _full((1, D)), _full((1, D))],
        out_specs=_rows((NB, D)),
        out_shape=jax.ShapeDtypeStruct((N_NODES, D), jnp.float32),
    )(h, pa, pb, w1, b1, w2, b2, w3, b3, g, beta)


def _sc_gather(h_nodes, idx_flat, n_e):
    ept = n_e // NS
    gnch = ept // GC

    @functools.partial(
        pl.kernel,
        out_type=jax.ShapeDtypeStruct((2, n_e, D), jnp.float32),
        mesh=_sc_mesh(),
        compiler_params=_SC_PARAMS,
        scratch_types=[
            pltpu.VMEM((GC,), jnp.int32),
            pltpu.VMEM((GC,), jnp.int32),
            pltpu.VMEM((GC, D), jnp.float32),
            pltpu.VMEM((GC, D), jnp.float32),
            pltpu.VMEM((GC, D), jnp.float32),
            pltpu.VMEM_SHARED((N_NODES, D), jnp.float32),
            pltpu.SemaphoreType.DMA,
            pltpu.SemaphoreType.DMA,
            pltpu.SemaphoreType.DMA,
        ],
    )
    def k(tbl_hbm, idx_hbm, g_hbm,
          i0_v, i1_v, r0_v, r1_v, stage_v, tbl_sh, gsem, osem0, osem1):
        cid = lax.axis_index("c")
        sid = lax.axis_index("s")

        def tload(j, carry):
            c = sid + j * NS

            @pl.when(c < ROW_CHUNKS)
            def _():
                pltpu.sync_copy(tbl_hbm.at[pl.ds(c * GC, GC)], stage_v)
                pltpu.sync_copy(stage_v, tbl_sh.at[pl.ds(c * GC, GC)])
            return carry

        lax.fori_loop(0, (ROW_CHUNKS + NS - 1) // NS, tload, 0)
        plsc.subcore_barrier()

        bufs = ((i0_v, r0_v, osem0), (i1_v, r1_v, osem1))

        def chunk_body(iv, rv, osem, base):
            pltpu.sync_copy(idx_hbm.at[pl.ds(cid * n_e + base, GC)], iv)
            pltpu.async_copy(tbl_sh.at[iv], rv, gsem).wait()
            pltpu.async_copy(rv, g_hbm.at[cid, pl.ds(base, GC)], osem)

        def pair(j, carry):
            for bi, (iv, rv, osem) in enumerate(bufs):
                @pl.when(j > 0)
                def _():
                    pltpu.make_async_copy(
                        rv, g_hbm.at[cid, pl.ds(0, GC)], osem).wait()

                chunk_body(iv, rv, osem, sid * ept + (2 * j + bi) * GC)
            return carry

        lax.fori_loop(0, gnch // 2, pair, 0)
        if gnch % 2:
            iv, rv, osem = bufs[0]
            pltpu.make_async_copy(rv, g_hbm.at[cid, pl.ds(0, GC)], osem).wait()
            chunk_body(iv, rv, osem, sid * ept + (gnch - 1) * GC)
        for iv, rv, osem in bufs:
            pltpu.make_async_copy(rv, g_hbm.at[cid, pl.ds(0, GC)], osem).wait()

    return k(h_nodes, idx_flat)


SGC = 40


def _sc_scatter(u, ridx, zeros_blk, n_e):
    epw = n_e // NW
    nchunk = epw // SGC
    assert nchunk % 2 == 1 and epw % SGC == 0

    @functools.partial(
        pl.kernel,
        out_type=jax.ShapeDtypeStruct((NC, N_NODES, D), jnp.float32),
        mesh=_sc_mesh(),
        compiler_params=_SC_PARAMS,
        scratch_types=[
            pltpu.VMEM((SGC, D), jnp.float32),
            pltpu.VMEM((SGC,), jnp.int32),
            pltpu.VMEM((SGC, D), jnp.float32),
            pltpu.VMEM((SGC,), jnp.int32),
            pltpu.VMEM((GC, D), jnp.float32),
            pltpu.VMEM_SHARED((N_NODES, D), jnp.float32),
            pltpu.SemaphoreType.DMA,
            pltpu.SemaphoreType.DMA,
        ],
    )
    def k(u_hbm, ri_hbm, z_hbm, p_hbm, u_v, ri_v, u2_v, ri2_v, row_v, acc_sh,
          lsem0, lsem1):
        cid = lax.axis_index("c")
        sid = lax.axis_index("s")
        wid = sid * NC + cid

        pltpu.sync_copy(z_hbm, row_v)

        def zloop(j, carry):
            c = sid + j * NS

            @pl.when(c < ROW_CHUNKS)
            def _():
                pltpu.sync_copy(row_v, acc_sh.at[pl.ds(c * GC, GC)])
            return carry

        lax.fori_loop(0, (ROW_CHUNKS + NS - 1) // NS, zloop, 0)
        plsc.subcore_barrier()

        ebase = wid * epw
        pltpu.sync_copy(ri_hbm.at[pl.ds(ebase, SGC)], ri_v)
        pltpu.sync_copy(u_hbm.at[pl.ds(ebase, SGC)], u_v)

        def pair(j, carry):
            b1 = ebase + (2 * j + 1) * SGC
            c1u = pltpu.async_copy(u_hbm.at[pl.ds(b1, SGC)], u2_v, lsem0)
            c1r = pltpu.async_copy(ri_hbm.at[pl.ds(b1, SGC)], ri2_v, lsem1)
            pltpu.sync_copy(u_v, acc_sh.at[ri_v], add=True)
            c1u.wait()
            c1r.wait()
            b2 = ebase + (2 * j + 2) * SGC
            c2u = pltpu.async_copy(u_hbm.at[pl.ds(b2, SGC)], u_v, lsem0)
            c2r = pltpu.async_copy(ri_hbm.at[pl.ds(b2, SGC)], ri_v, lsem1)
            pltpu.sync_copy(u2_v, acc_sh.at[ri2_v], add=True)
            c2u.wait()
            c2r.wait()
            return carry

        lax.fori_loop(0, (nchunk - 1) // 2, pair, 0)
        pltpu.sync_copy(u_v, acc_sh.at[ri_v], add=True)
        plsc.subcore_barrier()

        def oloop(j, carry):
            c = sid + j * NS

            @pl.when(c < ROW_CHUNKS)
            def _():
                pltpu.sync_copy(acc_sh.at[pl.ds(c * GC, GC)], row_v)
                pltpu.sync_copy(row_v, p_hbm.at[cid, pl.ds(c * GC, GC)])
            return carry

        lax.fori_loop(0, (ROW_CHUNKS + NS - 1) // NS, oloop, 0)

    return k(u, ridx, zeros_blk)


EH = N_EDGES // 2


def kernel(node_features, edge_features, edge_index, params):
    idx2 = edge_index.astype(jnp.int32)
    sender, receiver = idx2[0], idx2[1]
    idx_h = [jnp.concatenate([sender[h * EH:(h + 1) * EH],
                              receiver[h * EH:(h + 1) * EH]]) for h in (0, 1)]
    ridx_h = [receiver[h * EH:(h + 1) * EH] for h in (0, 1)]
    zeros_blk = jnp.zeros((GC, D), jnp.float32)

    h_nodes = _mlp_ln(node_features, params["node_enc"], NB)
    h_edges_full = _mlp_ln(edge_features, params["edge_enc"], EB)
    he_init = (h_edges_full[:EH], h_edges_full[EH:])

    layers = params["layers"]

    def stack(fn):
        return jnp.stack([fn(lp) for lp in layers])

    lw = {
        "ew1": stack(lambda lp: lp["edge_mlp"]["fc1"]["w"]),
        "eb1": stack(lambda lp: lp["edge_mlp"]["fc1"]["b"]),
        "ew2": stack(lambda lp: lp["edge_mlp"]["fc2"]["w"]),
        "eb2": stack(lambda lp: lp["edge_mlp"]["fc2"]["b"]),
        "ew3": stack(lambda lp: lp["edge_mlp"]["fc3"]["w"]),
        "eb3": stack(lambda lp: lp["edge_mlp"]["fc3"]["b"]),
        "eg": stack(lambda lp: lp["edge_mlp"]["ln"]["g"]),
        "ebt": stack(lambda lp: lp["edge_mlp"]["ln"]["b"]),
        "nw1": stack(lambda lp: lp["node_mlp"]["fc1"]["w"]),
        "nb1": stack(lambda lp: lp["node_mlp"]["fc1"]["b"]),
        "nw2": stack(lambda lp: lp["node_mlp"]["fc2"]["w"]),
        "nb2": stack(lambda lp: lp["node_mlp"]["fc2"]["b"]),
        "nw3": stack(lambda lp: lp["node_mlp"]["fc3"]["w"]),
        "nb3": stack(lambda lp: lp["node_mlp"]["fc3"]["b"]),
        "ng": stack(lambda lp: lp["node_mlp"]["ln"]["g"]),
        "nbt": stack(lambda lp: lp["node_mlp"]["ln"]["b"]),
    }

    def step(carry, w):
        h_nodes, he0, he1 = carry
        eargs = (w["ew1"], w["eb1"][None, :], w["ew2"], w["eb2"][None, :],
                 w["ew3"], w["eb3"][None, :], w["eg"][None, :],
                 w["ebt"][None, :])
        g0 = _sc_gather(h_nodes, idx_h[0], EH)
        g1 = _sc_gather(h_nodes, idx_h[1], EH)
        he0, u0 = _edge_mlp(he0, g0, *eargs)
        he1, u1 = _edge_mlp(he1, g1, *eargs)
        pa = _sc_scatter(u0, ridx_h[0], zeros_blk, EH)
        pb = _sc_scatter(u1, ridx_h[1], zeros_blk, EH)
        h_nodes = _node_mlp(h_nodes, pa, pb, w["nw1"], w["nb1"][None, :],
                            w["nw2"], w["nb2"][None, :], w["nw3"],
                            w["nb3"][None, :], w["ng"][None, :],
                            w["nbt"][None, :])
        return (h_nodes, he0, he1), None

    (h_nodes, _, _), _ = lax.scan(step, (h_nodes,) + he_init, lw)
    return _decoder(h_nodes, params["decoder"], NB)


# speedup vs baseline: 1.1344x; 1.0400x over previous
import functools

import jax
import jax.numpy as jnp
from jax import lax
from jax.experimental import pallas as pl
from jax.experimental.pallas import tpu as pltpu
from jax.experimental.pallas import tpu_sc as plsc

N_NODES = 10000
N_EDGES = 320000
D = 128

NC = 2
NS = 16
NW = NC * NS
GC = 80
ROW_CHUNKS = N_NODES // GC

EB = 2000
NB = 2000

_SC_PARAMS = None


def _sc_mesh():
    return plsc.VectorSubcoreMesh(core_axis_name="c", subcore_axis_name="s",
                                  num_cores=NC, num_subcores=NS)


def _ln(y, g, b):
    mu = jnp.mean(y, axis=-1, keepdims=True)
    yc = y - mu
    var = jnp.mean(yc * yc, axis=-1, keepdims=True)
    return yc / jnp.sqrt(var + 1e-5) * g + b


def _dot(a, b):
    return jnp.dot(a, b, preferred_element_type=jnp.float32)


def _full(shape):
    return pl.BlockSpec(shape, lambda i: tuple(0 for _ in shape))


def _rows(shape):
    return pl.BlockSpec(shape, lambda i: (i, 0))


def _mlp_ln_kernel(x_ref, w1, b1, w2, b2, w3, b3, g, beta, o_ref):
    x1 = jax.nn.relu(_dot(x_ref[...], w1[...]) + b1[...])
    x2 = jax.nn.relu(_dot(x1, w2[...]) + b2[...])
    y = _dot(x2, w3[...]) + b3[...]
    o_ref[...] = _ln(y, g[...], beta[...])


def _mlp_ln(x, p, block_rows):
    n, din = x.shape
    w1, w2, w3 = p["fc1"]["w"], p["fc2"]["w"], p["fc3"]["w"]
    dout = w3.shape[1]
    args = (x, w1, p["fc1"]["b"][None, :], w2, p["fc2"]["b"][None, :],
            w3, p["fc3"]["b"][None, :], p["ln"]["g"][None, :], p["ln"]["b"][None, :])
    return pl.pallas_call(
        _mlp_ln_kernel,
        grid=(n // block_rows,),
        in_specs=[_rows((block_rows, din)), _full(w1.shape), _full((1, D)),
                  _full(w2.shape), _full((1, D)), _full(w3.shape), _full((1, dout)),
                  _full((1, dout)), _full((1, dout))],
        out_specs=_rows((block_rows, dout)),
        out_shape=jax.ShapeDtypeStruct((n, dout), jnp.float32),
    )(*args)


def _dec_kernel(x_ref, w1, b1, w2, b2, w3, b3, o_ref):
    x1 = jax.nn.relu(_dot(x_ref[...], w1[...]) + b1[...])
    x2 = jax.nn.relu(_dot(x1, w2[...]) + b2[...])
    o_ref[...] = _dot(x2, w3[...]) + b3[...]


def _decoder(x, p, block_rows):
    n, din = x.shape
    dout = p["fc3"]["w"].shape[1]
    return pl.pallas_call(
        _dec_kernel,
        grid=(n // block_rows,),
        in_specs=[_rows((block_rows, din)), _full((din, D)), _full((1, D)),
                  _full((D, D)), _full((1, D)), _full((D, dout)), _full((1, dout))],
        out_specs=_rows((block_rows, dout)),
        out_shape=jax.ShapeDtypeStruct((n, dout), jnp.float32),
    )(x, p["fc1"]["w"], p["fc1"]["b"][None, :], p["fc2"]["w"], p["fc2"]["b"][None, :],
      p["fc3"]["w"], p["fc3"]["b"][None, :])


def _pre_proj_kernel(h_ref, w_ref, ab_ref):
    ab_ref[0] = _dot(h_ref[...], w_ref[0])


def _pre_proj(h, wsr):
    return pl.pallas_call(
        _pre_proj_kernel,
        grid=(2, N_NODES // NB),
        in_specs=[pl.BlockSpec((NB, D), lambda i, j: (j, 0)),
                  pl.BlockSpec((1, D, D), lambda i, j: (i, 0, 0))],
        out_specs=pl.BlockSpec((1, NB, D), lambda i, j: (i, j, 0)),
        out_shape=jax.ShapeDtypeStruct((2, N_NODES, D), jnp.float32),
    )(h, wsr)


def _edge_mlp_kernel(he_ref, g2_ref, w1e, b1, w2, b2, w3, b3, g, beta,
                     ho_ref, u_ref):
    he = he_ref[...]
    x1 = jax.nn.relu(_dot(he, w1e[...]) + g2_ref[0] + g2_ref[1] + b1[...])
    x2 = jax.nn.relu(_dot(x1, w2[...]) + b2[...])
    y = _dot(x2, w3[...]) + b3[...]
    u = _ln(y, g[...], beta[...])
    u_ref[...] = u
    ho_ref[...] = he + u


def _edge_mlp(h_edges, gsr, w1, b1, w2, b2, w3, b3, g, beta):
    n_e = h_edges.shape[0]
    return pl.pallas_call(
        _edge_mlp_kernel,
        grid=(n_e // EB,),
        in_specs=[_rows((EB, D)),
                  pl.BlockSpec((2, EB, D), lambda i: (0, i, 0)),
                  _full((D, D)), _full((1, D)), _full((D, D)), _full((1, D)),
                  _full((D, D)), _full((1, D)), _full((1, D)), _full((1, D))],
        out_specs=(_rows((EB, D)), _rows((EB, D))),
        out_shape=(jax.ShapeDtypeStruct((n_e, D), jnp.float32),
                   jax.ShapeDtypeStruct((n_e, D), jnp.float32)),
    )(h_edges, gsr, w1, b1, w2, b2, w3, b3, g, beta)


def _node_mlp_kernel(h_ref, pa_ref, pb_ref, w1, b1, w2, b2, w3, b3, g, beta,
                     ho_ref):
    h = h_ref[...]
    agg = pa_ref[0] + pa_ref[1] + pb_ref[0] + pb_ref[1]
    x = jnp.concatenate([h, agg], axis=-1)
    x1 = jax.nn.relu(_dot(x, w1[...]) + b1[...])
    x2 = jax.nn.relu(_dot(x1, w2[...]) + b2[...])
    y = _dot(x2, w3[...]) + b3[...]
    ho_ref[...] = h + _ln(y, g[...], beta[...])


def _node_mlp(h, pa, pb, w1, b1, w2, b2, w3, b3, g, beta):
    p2 = pl.BlockSpec((2, NB, D), lambda i: (0, i, 0))
    return pl.pallas_call(
        _node_mlp_kernel,
        grid=(N_NODES // NB,),
        in_specs=[_rows((NB, D)), p2, p2,
                  _full((2 * D, D)), _full((1, D)), _full((D, D)), _full((1, D)),
                  _full((D, D)), _full((1, D)), _full((1, D)), _full((1, D))],
        out_specs=_rows((NB, D)),
        out_shape=jax.ShapeDtypeStruct((N_NODES, D), jnp.float32),
    )(h, pa, pb, w1, b1, w2, b2, w3, b3, g, beta)


def _sc_gather(ab, idx_flat, n_e):
    ept = n_e // NS
    gnch = ept // GC

    @functools.partial(
        pl.kernel,
        out_type=jax.ShapeDtypeStruct((2, n_e, D), jnp.float32),
        mesh=_sc_mesh(),
        compiler_params=_SC_PARAMS,
        scratch_types=[
            pltpu.VMEM((GC,), jnp.int32),
            pltpu.VMEM((GC,), jnp.int32),
            pltpu.VMEM((GC, D), jnp.float32),
            pltpu.VMEM((GC, D), jnp.float32),
            pltpu.VMEM((GC, D), jnp.float32),
            pltpu.VMEM_SHARED((N_NODES, D), jnp.float32),
            pltpu.SemaphoreType.DMA,
            pltpu.SemaphoreType.DMA,
            pltpu.SemaphoreType.DMA,
        ],
    )
    def k(ab_hbm, idx_hbm, g_hbm,
          i0_v, i1_v, r0_v, r1_v, stage_v, tbl_sh, gsem, osem0, osem1):
        cid = lax.axis_index("c")
        sid = lax.axis_index("s")

        def tload(j, carry):
            c = sid + j * NS

            @pl.when(c < ROW_CHUNKS)
            def _():
                pltpu.sync_copy(ab_hbm.at[cid, pl.ds(c * GC, GC)], stage_v)
                pltpu.sync_copy(stage_v, tbl_sh.at[pl.ds(c * GC, GC)])
            return carry

        lax.fori_loop(0, (ROW_CHUNKS + NS - 1) // NS, tload, 0)
        plsc.subcore_barrier()

        bufs = ((i0_v, r0_v, osem0), (i1_v, r1_v, osem1))

        def chunk_body(iv, rv, osem, base):
            pltpu.sync_copy(idx_hbm.at[pl.ds(cid * n_e + base, GC)], iv)
            pltpu.async_copy(tbl_sh.at[iv], rv, gsem).wait()
            pltpu.async_copy(rv, g_hbm.at[cid, pl.ds(base, GC)], osem)

        def pair(j, carry):
            for bi, (iv, rv, osem) in enumerate(bufs):
                @pl.when(j > 0)
                def _():
                    pltpu.make_async_copy(
                        rv, g_hbm.at[cid, pl.ds(0, GC)], osem).wait()

                chunk_body(iv, rv, osem, sid * ept + (2 * j + bi) * GC)
            return carry

        lax.fori_loop(0, gnch // 2, pair, 0)
        if gnch % 2:
            iv, rv, osem = bufs[0]
            pltpu.make_async_copy(rv, g_hbm.at[cid, pl.ds(0, GC)], osem).wait()
            chunk_body(iv, rv, osem, sid * ept + (gnch - 1) * GC)
        for iv, rv, osem in bufs:
            pltpu.make_async_copy(rv, g_hbm.at[cid, pl.ds(0, GC)], osem).wait()

    return k(ab, idx_flat)


SGC = 40


def _sc_scatter(u, ridx, zeros_blk, n_e):
    epw = n_e // NW
    nchunk = epw // SGC
    assert nchunk % 2 == 1 and epw % SGC == 0

    @functools.partial(
        pl.kernel,
        out_type=jax.ShapeDtypeStruct((NC, N_NODES, D), jnp.float32),
        mesh=_sc_mesh(),
        compiler_params=_SC_PARAMS,
        scratch_types=[
            pltpu.VMEM((SGC, D), jnp.float32),
            pltpu.VMEM((SGC,), jnp.int32),
            pltpu.VMEM((SGC, D), jnp.float32),
            pltpu.VMEM((SGC,), jnp.int32),
            pltpu.VMEM((GC, D), jnp.float32),
            pltpu.VMEM_SHARED((N_NODES, D), jnp.float32),
            pltpu.SemaphoreType.DMA,
            pltpu.SemaphoreType.DMA,
        ],
    )
    def k(u_hbm, ri_hbm, z_hbm, p_hbm, u_v, ri_v, u2_v, ri2_v, row_v, acc_sh,
          lsem0, lsem1):
        cid = lax.axis_index("c")
        sid = lax.axis_index("s")
        wid = sid * NC + cid

        pltpu.sync_copy(z_hbm, row_v)

        def zloop(j, carry):
            c = sid + j * NS

            @pl.when(c < ROW_CHUNKS)
            def _():
                pltpu.sync_copy(row_v, acc_sh.at[pl.ds(c * GC, GC)])
            return carry

        lax.fori_loop(0, (ROW_CHUNKS + NS - 1) // NS, zloop, 0)
        plsc.subcore_barrier()

        ebase = wid * epw
        pltpu.sync_copy(ri_hbm.at[pl.ds(ebase, SGC)], ri_v)
        pltpu.sync_copy(u_hbm.at[pl.ds(ebase, SGC)], u_v)

        def pair(j, carry):
            b1 = ebase + (2 * j + 1) * SGC
            c1u = pltpu.async_copy(u_hbm.at[pl.ds(b1, SGC)], u2_v, lsem0)
            c1r = pltpu.async_copy(ri_hbm.at[pl.ds(b1, SGC)], ri2_v, lsem1)
            pltpu.sync_copy(u_v, acc_sh.at[ri_v], add=True)
            c1u.wait()
            c1r.wait()
            b2 = ebase + (2 * j + 2) * SGC
            c2u = pltpu.async_copy(u_hbm.at[pl.ds(b2, SGC)], u_v, lsem0)
            c2r = pltpu.async_copy(ri_hbm.at[pl.ds(b2, SGC)], ri_v, lsem1)
            pltpu.sync_copy(u2_v, acc_sh.at[ri2_v], add=True)
            c2u.wait()
            c2r.wait()
            return carry

        lax.fori_loop(0, (nchunk - 1) // 2, pair, 0)
        pltpu.sync_copy(u_v, acc_sh.at[ri_v], add=True)
        plsc.subcore_barrier()

        def oloop(j, carry):
            c = sid + j * NS

            @pl.when(c < ROW_CHUNKS)
            def _():
                pltpu.sync_copy(acc_sh.at[pl.ds(c * GC, GC)], row_v)
                pltpu.sync_copy(row_v, p_hbm.at[cid, pl.ds(c * GC, GC)])
            return carry

        lax.fori_loop(0, (ROW_CHUNKS + NS - 1) // NS, oloop, 0)

    return k(u, ridx, zeros_blk)


EH = N_EDGES // 2


def kernel(node_features, edge_features, edge_index, params):
    idx2 = edge_index.astype(jnp.int32)
    sender, receiver = idx2[0], idx2[1]
    idx_h = [jnp.concatenate([sender[h * EH:(h + 1) * EH],
                              receiver[h * EH:(h + 1) * EH]]) for h in (0, 1)]
    ridx_h = [receiver[h * EH:(h + 1) * EH] for h in (0, 1)]
    zeros_blk = jnp.zeros((GC, D), jnp.float32)

    h_nodes = _mlp_ln(node_features, params["node_enc"], NB)
    h_edges_full = _mlp_ln(edge_features, params["edge_enc"], EB)
    he_init = (h_edges_full[:EH], h_edges_full[EH:])

    layers = params["layers"]

    def stack(fn):
        return jnp.stack([fn(lp) for lp in layers])

    lw = {
        "ew1": stack(lambda lp: lp["edge_mlp"]["fc1"]["w"]),
        "eb1": stack(lambda lp: lp["edge_mlp"]["fc1"]["b"]),
        "ew2": stack(lambda lp: lp["edge_mlp"]["fc2"]["w"]),
        "eb2": stack(lambda lp: lp["edge_mlp"]["fc2"]["b"]),
        "ew3": stack(lambda lp: lp["edge_mlp"]["fc3"]["w"]),
        "eb3": stack(lambda lp: lp["edge_mlp"]["fc3"]["b"]),
        "eg": stack(lambda lp: lp["edge_mlp"]["ln"]["g"]),
        "ebt": stack(lambda lp: lp["edge_mlp"]["ln"]["b"]),
        "nw1": stack(lambda lp: lp["node_mlp"]["fc1"]["w"]),
        "nb1": stack(lambda lp: lp["node_mlp"]["fc1"]["b"]),
        "nw2": stack(lambda lp: lp["node_mlp"]["fc2"]["w"]),
        "nb2": stack(lambda lp: lp["node_mlp"]["fc2"]["b"]),
        "nw3": stack(lambda lp: lp["node_mlp"]["fc3"]["w"]),
        "nb3": stack(lambda lp: lp["node_mlp"]["fc3"]["b"]),
        "ng": stack(lambda lp: lp["node_mlp"]["ln"]["g"]),
        "nbt": stack(lambda lp: lp["node_mlp"]["ln"]["b"]),
    }

    def step(carry, w):
        h_nodes, he0, he1 = carry
        eargs = (w["ew1"][:D], w["eb1"][None, :], w["ew2"], w["eb2"][None, :],
                 w["ew3"], w["eb3"][None, :], w["eg"][None, :],
                 w["ebt"][None, :])
        wsr = jnp.stack([w["ew1"][D:2 * D], w["ew1"][2 * D:]])
        ab = _pre_proj(h_nodes, wsr)
        g0 = _sc_gather(ab, idx_h[0], EH)
        g1 = _sc_gather(ab, idx_h[1], EH)
        he0, u0 = _edge_mlp(he0, g0, *eargs)
        he1, u1 = _edge_mlp(he1, g1, *eargs)
        pa = _sc_scatter(u0, ridx_h[0], zeros_blk, EH)
        pb = _sc_scatter(u1, ridx_h[1], zeros_blk, EH)
        h_nodes = _node_mlp(h_nodes, pa, pb, w["nw1"], w["nb1"][None, :],
                            w["nw2"], w["nb2"][None, :], w["nw3"],
                            w["nb3"][None, :], w["ng"][None, :],
                            w["nbt"][None, :])
        return (h_nodes, he0, he1), None

    (h_nodes, _, _), _ = lax.scan(step, (h_nodes,) + he_init, lw)
    return _decoder(h_nodes, params["decoder"], NB)
